# baseline (speedup 1.0000x reference)
import dataclasses
import functools

import jax
import jax.numpy as jnp
import numpy as np
from jax import lax
from jax.experimental import pallas as pl
from jax.experimental.pallas import tpu as pltpu
from jax.experimental.pallas import tpu_sc as plsc

MAX_POS = 200
EMBED_DIM = 64
LANES = 16

NUM_CORES = 2
NUM_SUBCORES = 16
NUM_WORKERS = NUM_CORES * NUM_SUBCORES

BLK = 128
NBUF = 8
GBUF = 4
DT = EMBED_DIM // 8


def _pos_enc() -> np.ndarray:
    pos = np.arange(MAX_POS)[:, np.newaxis]
    i = np.arange(EMBED_DIM)[np.newaxis, :]
    angle_rates = 1 / np.power(10000, 2 * (i // 2) / np.float32(EMBED_DIM))
    angle_rads = pos * angle_rates
    angle_rads[:, 0::2] = np.sin(angle_rads[:, 0::2])
    angle_rads[:, 1::2] = np.cos(angle_rads[:, 1::2])
    return angle_rads.astype(np.float32)


def _make_sc_embed(batch: int, seq: int):
    assert batch == BLK * NUM_WORKERS and seq % NBUF == 0
    mesh = plsc.VectorSubcoreMesh(core_axis_name="c", subcore_axis_name="s")
    cp = pltpu.CompilerParams()
    if "needs_layout_passes" in pltpu.CompilerParams.__dataclass_fields__:
        cp = dataclasses.replace(cp, needs_layout_passes=False)
    if "use_tc_tiling_on_sc" in pltpu.CompilerParams.__dataclass_fields__:
        cp = dataclasses.replace(cp, use_tc_tiling_on_sc=False)
    if "disable_bounds_checks" in pltpu.CompilerParams.__dataclass_fields__:
        cp = dataclasses.replace(cp, disable_bounds_checks=True)

    @functools.partial(
        pl.kernel,
        out_type=jax.ShapeDtypeStruct(
            (seq, DT, batch // BLK, 8, BLK), jnp.float32),
        mesh=mesh,
        compiler_params=cp,
        scratch_types=[
            pltpu.VMEM((NBUF, BLK), jnp.int32),
            pltpu.VMEM((NBUF, BLK), jnp.int32),
            pltpu.VMEM((NBUF, BLK), jnp.float32),
            pltpu.VMEM((GBUF, BLK, EMBED_DIM), jnp.float32),
            pltpu.VMEM((GBUF, DT, 8, BLK), jnp.float32),
            pltpu.VMEM((MAX_POS, EMBED_DIM), jnp.float32),
            pltpu.SemaphoreType.DMA((NBUF,)),
            pltpu.SemaphoreType.DMA((GBUF,)),
            pltpu.SemaphoreType.DMA((GBUF,)),
        ],
    )
    def sc_embed(table_hbm, xt_hbm, pos_hbm, out_hbm,
                 idx_v, duri_v, durf_v, g_v, t_v, pos_v,
                 sem_in, sem_g, sem_out):
        wid = lax.axis_index("s") * NUM_CORES + lax.axis_index("c")
        pltpu.sync_copy(pos_hbm, pos_v)

        def issue_in(s, b):
            pltpu.async_copy(
                xt_hbm.at[s, wid, 0], idx_v.at[b], sem_in.at[b])
            pltpu.async_copy(
                xt_hbm.at[s, wid, 1], duri_v.at[b], sem_in.at[b])

        def wait_in(b):
            for _ in range(2):
                pltpu.make_async_copy(
                    xt_hbm.at[0, 0, 0], idx_v.at[b], sem_in.at[b]).wait()

        GSPLIT, GROWS = 4, BLK // 4

        def issue_gather(b):
            for j in range(GSPLIT):
                pltpu.async_copy(
                    table_hbm.at[idx_v.at[b, pl.ds(j * GROWS, GROWS)]],
                    g_v.at[b % GBUF].at[pl.ds(j * GROWS, GROWS)],
                    sem_g.at[b % GBUF])

        def wait_gather(b):
            for j in range(GSPLIT):
                pltpu.make_async_copy(
                    table_hbm.at[idx_v.at[b, pl.ds(j * GROWS, GROWS)]],
                    g_v.at[b % GBUF].at[pl.ds(j * GROWS, GROWS)],
                    sem_g.at[b % GBUF]).wait()

        def issue_out(s, b):
            pltpu.async_copy(t_v.at[b], out_hbm.at[s, :, wid], sem_out.at[b])

        def wait_out(b):
            pltpu.make_async_copy(
                t_v.at[b], out_hbm.at[0, :, wid], sem_out.at[b]).wait()

        dvals = [lax.iota(jnp.int32, LANES) + k * LANES
                 for k in range(EMBED_DIM // LANES)]
        dts = [d >> 3 for d in dvals]
        dins = [d & 7 for d in dvals]

        for k in range(4):
            issue_in(k, k)
        for k in range(2):
            wait_in(k)
            issue_gather(k)

        @pl.loop(0, seq, step=NBUF)
        def _ring(cc):
            for b in range(NBUF):
                c = cc + b
                g = b % GBUF
                b2, b4 = (b + 2) % NBUF, (b + 4) % NBUF

                @pl.when(c + 2 < seq)
                def _():
                    wait_in(b2)
                    issue_gather(b2)

                @pl.when(c + 4 < seq)
                def _():
                    issue_in(c + 4, b4)

                wait_gather(b)

                @pl.when(c >= GBUF)
                def _():
                    wait_out(g)

                for k in range(BLK // LANES):
                    sl = pl.ds(k * LANES, LANES)
                    durf_v[b, sl] = duri_v[b, sl].astype(jnp.float32)

                pvs = [pos_v[c, pl.ds(k * LANES, LANES)]
                       for k in range(EMBED_DIM // LANES)]

                @plsc.parallel_loop(0, BLK, 1, unroll=2)
                def _tok(t):
                    tb = lax.broadcast(t, (LANES,))
                    durb = plsc.load_gather(durf_v.at[b], [tb])
                    for k in range(EMBED_DIM // LANES):
                        v = g_v[g, t, pl.ds(k * LANES, LANES)] + pvs[k] + durb
                        plsc.store_scatter(
                            t_v.at[g], [dts[k], dins[k], tb], v)

                issue_out(c, g)

        for k in range(seq - GBUF, seq):
            wait_out(k % GBUF)

    return sc_embed


def kernel(x_in, table):
    batch, seq, _ = x_in.shape
    xt = (x_in.transpose((1, 0, 2))
          .reshape(seq, batch // BLK, BLK, 2)
          .transpose((0, 1, 3, 2)))
    pos = jnp.asarray(_pos_enc())
    x5 = _make_sc_embed(batch, seq)(table, xt, pos)
    return x5.transpose((2, 4, 0, 1, 3)).reshape(batch, seq, EMBED_DIM)


# speedup vs baseline: 3.4314x; 3.4314x over previous
import dataclasses
import functools

import jax
import jax.numpy as jnp
import numpy as np
from jax import lax
from jax.experimental import pallas as pl
from jax.experimental.pallas import tpu as pltpu
from jax.experimental.pallas import tpu_sc as plsc

MAX_POS = 200
EMBED_DIM = 64
LANES = 16

NUM_CORES = 2
NUM_SUBCORES = 16
NUM_WORKERS = NUM_CORES * NUM_SUBCORES

BLK = 128
NBUF = 8
GBUF = 4
DT = EMBED_DIM // 8


def _pos_enc() -> np.ndarray:
    pos = np.arange(MAX_POS)[:, np.newaxis]
    i = np.arange(EMBED_DIM)[np.newaxis, :]
    angle_rates = 1 / np.power(10000, 2 * (i // 2) / np.float32(EMBED_DIM))
    angle_rads = pos * angle_rates
    angle_rads[:, 0::2] = np.sin(angle_rads[:, 0::2])
    angle_rads[:, 1::2] = np.cos(angle_rads[:, 1::2])
    return angle_rads.astype(np.float32)


def _make_sc_embed(batch: int, seq: int):
    assert batch == BLK * NUM_WORKERS and seq % NBUF == 0
    mesh = plsc.VectorSubcoreMesh(core_axis_name="c", subcore_axis_name="s")
    cp = pltpu.CompilerParams()
    if "needs_layout_passes" in pltpu.CompilerParams.__dataclass_fields__:
        cp = dataclasses.replace(cp, needs_layout_passes=False)
    if "use_tc_tiling_on_sc" in pltpu.CompilerParams.__dataclass_fields__:
        cp = dataclasses.replace(cp, use_tc_tiling_on_sc=False)
    if "disable_bounds_checks" in pltpu.CompilerParams.__dataclass_fields__:
        cp = dataclasses.replace(cp, disable_bounds_checks=True)

    @functools.partial(
        pl.kernel,
        out_type=jax.ShapeDtypeStruct(
            (seq, DT, batch // BLK, 8, BLK), jnp.float32),
        mesh=mesh,
        compiler_params=cp,
        scratch_types=[
            pltpu.VMEM((NBUF, BLK), jnp.int32),
            pltpu.VMEM((NBUF, BLK), jnp.int32),
            pltpu.VMEM((NBUF, BLK), jnp.float32),
            pltpu.VMEM((GBUF, BLK, EMBED_DIM), jnp.float32),
            pltpu.VMEM((GBUF, DT, 8, BLK + 1), jnp.float32),
            pltpu.VMEM((MAX_POS, EMBED_DIM), jnp.float32),
            pltpu.SemaphoreType.DMA((NBUF,)),
            pltpu.SemaphoreType.DMA((GBUF,)),
            pltpu.SemaphoreType.DMA((GBUF,)),
        ],
    )
    def sc_embed(table_hbm, xt_hbm, pos_hbm, out_hbm,
                 idx_v, duri_v, durf_v, g_v, t_v, pos_v,
                 sem_in, sem_g, sem_out):
        wid = lax.axis_index("s") * NUM_CORES + lax.axis_index("c")
        pltpu.sync_copy(pos_hbm, pos_v)

        def issue_in(s, b):
            pltpu.async_copy(
                xt_hbm.at[s, wid, 0], idx_v.at[b], sem_in.at[b])
            pltpu.async_copy(
                xt_hbm.at[s, wid, 1], duri_v.at[b], sem_in.at[b])

        def wait_in(b):
            for _ in range(2):
                pltpu.make_async_copy(
                    xt_hbm.at[0, 0, 0], idx_v.at[b], sem_in.at[b]).wait()

        GSPLIT, GROWS = 4, BLK // 4

        def issue_gather(b):
            for j in range(GSPLIT):
                pltpu.async_copy(
                    table_hbm.at[idx_v.at[b, pl.ds(j * GROWS, GROWS)]],
                    g_v.at[b % GBUF].at[pl.ds(j * GROWS, GROWS)],
                    sem_g.at[b % GBUF])

        def wait_gather(b):
            for j in range(GSPLIT):
                pltpu.make_async_copy(
                    table_hbm.at[idx_v.at[b, pl.ds(j * GROWS, GROWS)]],
                    g_v.at[b % GBUF].at[pl.ds(j * GROWS, GROWS)],
                    sem_g.at[b % GBUF]).wait()

        def issue_out(s, b):
            pltpu.async_copy(
                t_v.at[b, :, :, pl.ds(0, BLK)], out_hbm.at[s, :, wid],
                sem_out.at[b])

        def wait_out(b):
            pltpu.make_async_copy(
                t_v.at[b, :, :, pl.ds(0, BLK)], out_hbm.at[0, :, wid],
                sem_out.at[b]).wait()

        dvals = [lax.iota(jnp.int32, LANES) + k * LANES
                 for k in range(EMBED_DIM // LANES)]
        dts = [d >> 3 for d in dvals]
        dins = [d & 7 for d in dvals]

        for k in range(4):
            issue_in(k, k)
        for k in range(2):
            wait_in(k)
            issue_gather(k)

        @pl.loop(0, seq, step=NBUF)
        def _ring(cc):
            for b in range(NBUF):
                c = cc + b
                g = b % GBUF
                b2, b4 = (b + 2) % NBUF, (b + 4) % NBUF

                @pl.when(c + 2 < seq)
                def _():
                    wait_in(b2)
                    issue_gather(b2)

                @pl.when(c + 4 < seq)
                def _():
                    issue_in(c + 4, b4)

                wait_gather(b)

                @pl.when(c >= GBUF)
                def _():
                    wait_out(g)

                for k in range(BLK // LANES):
                    sl = pl.ds(k * LANES, LANES)
                    durf_v[b, sl] = duri_v[b, sl].astype(jnp.float32)

                pvs = [pos_v[c, pl.ds(k * LANES, LANES)]
                       for k in range(EMBED_DIM // LANES)]

                @plsc.parallel_loop(0, BLK, 1, unroll=2)
                def _tok(t):
                    tb = lax.broadcast(t, (LANES,))
                    durb = plsc.load_gather(durf_v.at[b], [tb])
                    for k in range(EMBED_DIM // LANES):
                        v = g_v[g, t, pl.ds(k * LANES, LANES)] + pvs[k] + durb
                        plsc.store_scatter(
                            t_v.at[g], [dts[k], dins[k], tb], v)

                issue_out(c, g)

        for k in range(seq - GBUF, seq):
            wait_out(k % GBUF)

    return sc_embed


def kernel(x_in, table):
    batch, seq, _ = x_in.shape
    xt = (x_in.transpose((1, 0, 2))
          .reshape(seq, batch // BLK, BLK, 2)
          .transpose((0, 1, 3, 2)))
    pos = jnp.asarray(_pos_enc())
    x5 = _make_sc_embed(batch, seq)(table, xt, pos)
    return x5.transpose((2, 4, 0, 1, 3)).reshape(batch, seq, EMBED_DIM)
